# baseline (speedup 1.0000x reference)
import jax
import jax.numpy as jnp
from jax.experimental import pallas as pl

_BM = 1024
_KF = 32

_LOG2E = 1.4426950408889634


def _fused_body(f_ref, g_ref, v_ref, o_ref):
    s = jax.lax.dot_general(f_ref[...], g_ref[...],
                            (((0,), (0,)), ((), ())),
                            preferred_element_type=jnp.float32)
    w = jnp.exp2(s)
    o_ref[...] = jnp.dot(w, v_ref[...], preferred_element_type=jnp.float32)


def _split_hi_lo(x):
    hi = x.astype(jnp.bfloat16)
    lo = (x - hi.astype(jnp.float32)).astype(jnp.bfloat16)
    return hi, lo


def kernel(means, values, covariances, conics, opacities, samples):
    del covariances
    M = samples.shape[0]
    N = means.shape[0]
    C = values.shape[1]
    l2e = jnp.float32(_LOG2E)

    A11, A12, A13, A22, A23, A33 = [conics[:, i] for i in range(6)]
    mx, my, mz = means[:, 0], means[:, 1], means[:, 2]
    bx = A11 * mx + A12 * my + A13 * mz
    by = A12 * mx + A22 * my + A23 * mz
    bz = A13 * mx + A23 * my + A33 * mz
    c = mx * bx + my * by + mz * bz
    g9 = jnp.stack([-0.5 * A11, -0.5 * A12, -0.5 * A13,
                    -0.5 * A22, -0.5 * A23, -0.5 * A33,
                    bx, by, bz], axis=0) * l2e
    cs = (-0.5 * c) * l2e
    g_hi, g_lo = _split_hi_lo(g9)
    c_hi, c_lo = _split_hi_lo(cs)
    g_mat = jnp.concatenate(
        [g_hi, c_hi[None, :], g_lo, c_lo[None, :], g_hi,
         jnp.zeros((3, N), jnp.bfloat16)], axis=0)

    sx, sy, sz = samples[:, 0], samples[:, 1], samples[:, 2]
    f9 = jnp.stack([sx * sx, 2.0 * sx * sy, 2.0 * sx * sz,
                    sy * sy, 2.0 * sy * sz, sz * sz,
                    sx, sy, sz], axis=0)
    f_hi, f_lo = _split_hi_lo(f9)
    ones = jnp.ones((1, M), jnp.bfloat16)
    f_mat = jnp.concatenate(
        [f_hi, ones, f_hi, ones, f_lo,
         jnp.zeros((3, M), jnp.bfloat16)], axis=0)

    v_mat = opacities * values

    out = pl.pallas_call(
        _fused_body,
        grid=(M // _BM,),
        in_specs=[
            pl.BlockSpec((_KF, _BM), lambda i: (0, i)),
            pl.BlockSpec((_KF, N), lambda i: (0, 0)),
            pl.BlockSpec((N, C), lambda i: (0, 0)),
        ],
        out_specs=pl.BlockSpec((_BM, C), lambda i: (i, 0)),
        out_shape=jax.ShapeDtypeStruct((M, C), jnp.float32),
    )(f_mat, g_mat, v_mat)
    return out


# speedup vs baseline: 1.2640x; 1.2640x over previous
import jax
import jax.numpy as jnp
from jax.experimental import pallas as pl

_BM = 1024
_KF = 32


def _fused_body(f_ref, g_ref, c_ref, v_ref, o_ref):
    s = jnp.dot(f_ref[...], g_ref[...], preferred_element_type=jnp.float32)
    s = s + c_ref[0:1, :]
    w = jnp.exp(s)
    o_ref[...] = jnp.dot(w, v_ref[...], preferred_element_type=jnp.float32)


def _hi(x):
    return x.astype(jnp.bfloat16).astype(jnp.float32)


def kernel(means, values, covariances, conics, opacities, samples):
    del covariances
    M = samples.shape[0]
    N = means.shape[0]
    C = values.shape[1]

    A11, A12, A13, A22, A23, A33 = [conics[:, i] for i in range(6)]
    mx, my, mz = means[:, 0], means[:, 1], means[:, 2]
    bx = A11 * mx + A12 * my + A13 * mz
    by = A12 * mx + A22 * my + A23 * mz
    bz = A13 * mx + A23 * my + A33 * mz
    c = mx * bx + my * by + mz * bz
    g9 = [-0.5 * A11, -0.5 * A12, -0.5 * A13,
          -0.5 * A22, -0.5 * A23, -0.5 * A33,
          bx, by, bz]
    zn = jnp.zeros((N,), jnp.float32)
    g_cols = ([_hi(x) for x in g9] + [x - _hi(x) for x in g9]
              + [_hi(x) for x in g9] + [zn] * 5)
    g_mat = jnp.stack(g_cols, axis=1).astype(jnp.bfloat16).T
    c_mat = jnp.broadcast_to((-0.5 * c)[None, :], (8, N))

    sx, sy, sz = samples[:, 0], samples[:, 1], samples[:, 2]
    f9 = [sx * sx, 2.0 * sx * sy, 2.0 * sx * sz,
          sy * sy, 2.0 * sy * sz, sz * sz,
          sx, sy, sz]
    zm = jnp.zeros((M,), jnp.float32)
    f_cols = ([_hi(x) for x in f9] + [_hi(x) for x in f9]
              + [x - _hi(x) for x in f9] + [zm] * 5)
    f_mat = jnp.stack(f_cols, axis=1).astype(jnp.bfloat16)

    v_mat = opacities * values

    out = pl.pallas_call(
        _fused_body,
        grid=(M // _BM,),
        in_specs=[
            pl.BlockSpec((_BM, _KF), lambda i: (i, 0)),
            pl.BlockSpec((_KF, N), lambda i: (0, 0)),
            pl.BlockSpec((8, N), lambda i: (0, 0)),
            pl.BlockSpec((N, C), lambda i: (0, 0)),
        ],
        out_specs=pl.BlockSpec((_BM, C), lambda i: (i, 0)),
        out_shape=jax.ShapeDtypeStruct((M, C), jnp.float32),
    )(f_mat, g_mat, c_mat, v_mat)
    return out
